# baseline (speedup 1.0000x reference)
import functools

import jax
import jax.numpy as jnp
from jax import lax
from jax.experimental import pallas as pl
from jax.experimental.pallas import tpu as pltpu
from jax.experimental.pallas import tpu_sc as plsc

_N = 10000
_E = 320000
_EPW = _E // 32
_K = 2
_BLK = _K * 128
_NFULL = _EPW // _BLK
_TAIL = _EPW - _NFULL * _BLK
_NACC = 10240
_BR = 2000


def _leaky(x):
    return jnp.where(x >= 0, x, 0.2 * x)


def _dense_in_body(x_ref, w_ref, asrc_ref, adst_ref, haug_ref, asad_ref):
    h = jnp.dot(x_ref[...], w_ref[...], preferred_element_type=jnp.float32)
    br = h.shape[0]
    haug_ref[...] = jnp.concatenate(
        [jnp.ones((br, 1), jnp.float32), jnp.zeros((br, 15), jnp.float32), h],
        axis=1)
    a_s = jnp.sum(h * asrc_ref[...], axis=1, keepdims=True)
    a_d = jnp.sum(h * adst_ref[...], axis=1, keepdims=True)
    asad_ref[...] = jnp.concatenate([a_s, a_d], axis=1)


def _combine_mid_body(acc_ref, haug_ref, asad_ref, b_ref, w_ref, asrc_ref,
                      adst_ref, haug2_ref, asad2_ref, *, d_in):
    acc = acc_ref[0] + acc_ref[1]
    asad = asad_ref[...]
    es = jnp.exp(_leaky(asad[:, 0] + asad[:, 1]))
    h1 = haug_ref[:, 16:16 + d_in]
    num = acc[:, 16:16 + d_in] + es[:, None] * h1
    den = acc[:, 0] + es
    g = num / (den[:, None] + 1e-16) + b_ref[...]
    hl = jnp.maximum(g, 0.0)
    h2 = jnp.dot(hl, w_ref[...], preferred_element_type=jnp.float32)
    br = h2.shape[0]
    haug2_ref[...] = jnp.concatenate(
        [jnp.ones((br, 1), jnp.float32), jnp.zeros((br, 15), jnp.float32), h2],
        axis=1)
    a_s = jnp.sum(h2 * asrc_ref[...], axis=1, keepdims=True)
    a_d = jnp.sum(h2 * adst_ref[...], axis=1, keepdims=True)
    asad2_ref[...] = jnp.concatenate([a_s, a_d], axis=1)


def _combine_out_body(acc_ref, haug_ref, asad_ref, b_ref, w_ref, fcb_ref,
                      emb_ref, out_ref, *, d_in):
    acc = acc_ref[0] + acc_ref[1]
    asad = asad_ref[...]
    es = jnp.exp(_leaky(asad[:, 0] + asad[:, 1]))
    h2 = haug_ref[:, 16:16 + d_in]
    num = acc[:, 16:16 + d_in] + es[:, None] * h2
    den = acc[:, 0] + es
    g = num / (den[:, None] + 1e-16) + b_ref[...]
    emb = jnp.maximum(g, 0.0)
    emb_ref[...] = emb
    out_ref[...] = jnp.dot(emb, w_ref[...],
                           preferred_element_type=jnp.float32) + fcb_ref[...]


def _sc_edge_body(edge_hbm, asad_hbm, haug_hbm, acc_hbm,
                  srcv, dstv, exv, rowsv, asadv, acc_sh, gsem, isem, ssem,
                  *, dp):
    c = lax.axis_index("c")
    s = lax.axis_index("s")
    wid = s * 2 + c
    e0 = wid * _EPW
    nj = dp // 16

    pltpu.sync_copy(asad_hbm, asadv.at[pl.ds(0, 2 * _N)])

    @plsc.parallel_loop(0, 128, unroll=8)
    def _z(r):
        for j in range(nj):
            rowsv[0, r, pl.ds(j * 16, 16)] = jnp.zeros((16,), jnp.float32)
    for z in range(5):
        pltpu.sync_copy(rowsv.at[0, pl.ds(0, 128)],
                        acc_sh.at[pl.ds(s * 640 + z * 128, 128)])
    plsc.subcore_barrier()

    def _issue_idx(t, b):
        rb = e0 + t * _BLK
        for k in range(_K):
            pltpu.async_copy(edge_hbm.at[0, pl.ds(rb + k * 128, 128)],
                             srcv.at[b, k], isem)
            pltpu.async_copy(edge_hbm.at[1, pl.ds(rb + k * 128, 128)],
                             dstv.at[b, k], isem)

    def _wait_idx(b):
        for _ in range(2 * _K):
            pltpu.make_async_copy(edge_hbm.at[0, pl.ds(0, 128)],
                                  srcv.at[b, 0], isem).wait()

    def _wait_scatters(nk=_K):
        for k in range(nk):
            pltpu.make_async_copy(haug_hbm.at[pl.ds(0, 128)],
                                  rowsv.at[0, pl.ds(0, 128)], ssem).wait()

    def _do_block(b, nk=_K):
        rows2 = rowsv.at[b]
        cps = [pltpu.async_copy(haug_hbm.at[srcv.at[b, k]],
                                rows2.at[pl.ds(k * 128, 128)], gsem)
               for k in range(nk)]
        for k in range(nk):
            @plsc.parallel_loop(0, 8, unroll=4)
            def _grp(i, k=k):
                sv = srcv[b, k, pl.ds(i * 16, 16)]
                dv = dstv[b, k, pl.ds(i * 16, 16)]
                a = plsc.load_gather(asadv, [sv * 2])
                ad = plsc.load_gather(asadv, [dv * 2 + 1])
                exv[pl.ds(k * 128 + i * 16, 16)] = jnp.exp(_leaky(a + ad))
        for cp in cps:
            cp.wait()

        @plsc.parallel_loop(0, nk * 128, unroll=8)
        def _scale(e):
            v = plsc.load_gather(exv, [jnp.full((16,), e, jnp.int32)])
            for j in range(nj):
                rows2[e, pl.ds(j * 16, 16)] = rows2[e, pl.ds(j * 16, 16)] * v

        for k in range(nk):
            pltpu.async_copy(rows2.at[pl.ds(k * 128, 128)],
                             acc_sh.at[dstv.at[b, k]], ssem, add=True)

    _issue_idx(0, 0)

    def _super(u, _):
        _wait_idx(0)

        @pl.when(u >= 1)
        def _():
            _wait_scatters()
        _issue_idx(2 * u + 1, 1)
        _do_block(0)

        _wait_idx(1)
        _wait_scatters()
        _issue_idx(2 * u + 2, 0)
        _do_block(1)
        return 0
    lax.fori_loop(0, (_NFULL - 1) // 2, _super, 0)

    _wait_idx(0)
    _wait_scatters()
    _do_block(0)

    pltpu.sync_copy(edge_hbm.at[0, pl.ds(e0 + _NFULL * _BLK, _TAIL)],
                    srcv.at[1, 0, pl.ds(0, _TAIL)])
    pltpu.sync_copy(edge_hbm.at[1, pl.ds(e0 + _NFULL * _BLK, _TAIL)],
                    dstv.at[1, 0, pl.ds(0, _TAIL)])
    iota16 = lax.iota(jnp.int32, 16)
    for g in range(_TAIL // 16, 8):
        srcv[1, 0, pl.ds(g * 16, 16)] = ((iota16 + g * 16) * 613) % _N
        dstv[1, 0, pl.ds(g * 16, 16)] = _N + ((iota16 + g * 16 + wid * 8)
                                              % (_NACC - _N))
    _wait_scatters()
    _do_block(1, nk=1)
    _wait_scatters(nk=1)
    plsc.subcore_barrier()

    pltpu.sync_copy(acc_sh.at[pl.ds(s * 640, 640)],
                    acc_hbm.at[c, pl.ds(s * 640, 640)])


def _sc_edge(edge_index, asad, haug, dp):
    mesh = plsc.VectorSubcoreMesh(core_axis_name="c", subcore_axis_name="s",
                                  num_cores=2, num_subcores=16)
    f = pl.kernel(
        functools.partial(_sc_edge_body, dp=dp),
        out_type=jax.ShapeDtypeStruct((2, _NACC, dp), jnp.float32),
        mesh=mesh,
        compiler_params=pltpu.CompilerParams(needs_layout_passes=False,
                                             use_tc_tiling_on_sc=False),
        scratch_types=[
            pltpu.VMEM((2, _K, 128), jnp.int32),
            pltpu.VMEM((2, _K, 128), jnp.int32),
            pltpu.VMEM((_BLK,), jnp.float32),
            pltpu.VMEM((2, _BLK, dp), jnp.float32),
            pltpu.VMEM((2 * _NACC,), jnp.float32),
            pltpu.VMEM_SHARED((_NACC, dp), jnp.float32),
            pltpu.SemaphoreType.DMA,
            pltpu.SemaphoreType.DMA,
            pltpu.SemaphoreType.DMA,
        ],
    )
    return f(edge_index, asad.reshape(-1), haug)


def _full_spec(shape):
    return pl.BlockSpec(shape, lambda i, _z=len(shape): (0,) * _z)


def kernel(x, edge_index, W1, a_src1, a_dst1, b1, W2, a_src2, a_dst2, b2,
           fcW, fcb):
    grid = _N // _BR

    haug1, asad1 = pl.pallas_call(
        _dense_in_body,
        grid=(grid,),
        in_specs=[
            pl.BlockSpec((_BR, 128), lambda i: (i, 0)),
            _full_spec((128, 64)),
            _full_spec((1, 64)),
            _full_spec((1, 64)),
        ],
        out_specs=[
            pl.BlockSpec((_BR, 80), lambda i: (i, 0)),
            pl.BlockSpec((_BR, 2), lambda i: (i, 0)),
        ],
        out_shape=[
            jax.ShapeDtypeStruct((_N, 80), jnp.float32),
            jax.ShapeDtypeStruct((_N, 2), jnp.float32),
        ],
    )(x, W1, a_src1.reshape(1, 64), a_dst1.reshape(1, 64))

    acc1 = _sc_edge(edge_index, asad1, haug1, 80)

    haug2, asad2 = pl.pallas_call(
        functools.partial(_combine_mid_body, d_in=64),
        grid=(grid,),
        in_specs=[
            pl.BlockSpec((2, _BR, 80), lambda i: (0, i, 0)),
            pl.BlockSpec((_BR, 80), lambda i: (i, 0)),
            pl.BlockSpec((_BR, 2), lambda i: (i, 0)),
            _full_spec((1, 64)),
            _full_spec((64, 32)),
            _full_spec((1, 32)),
            _full_spec((1, 32)),
        ],
        out_specs=[
            pl.BlockSpec((_BR, 48), lambda i: (i, 0)),
            pl.BlockSpec((_BR, 2), lambda i: (i, 0)),
        ],
        out_shape=[
            jax.ShapeDtypeStruct((_N, 48), jnp.float32),
            jax.ShapeDtypeStruct((_N, 2), jnp.float32),
        ],
    )(acc1, haug1, asad1, b1.reshape(1, 64), W2,
      a_src2.reshape(1, 32), a_dst2.reshape(1, 32))

    acc2 = _sc_edge(edge_index, asad2, haug2, 48)

    emb, out = pl.pallas_call(
        functools.partial(_combine_out_body, d_in=32),
        grid=(grid,),
        in_specs=[
            pl.BlockSpec((2, _BR, 48), lambda i: (0, i, 0)),
            pl.BlockSpec((_BR, 48), lambda i: (i, 0)),
            pl.BlockSpec((_BR, 2), lambda i: (i, 0)),
            _full_spec((1, 32)),
            _full_spec((32, 64)),
            _full_spec((1, 64)),
        ],
        out_specs=[
            pl.BlockSpec((_BR, 32), lambda i: (i, 0)),
            pl.BlockSpec((_BR, 64), lambda i: (i, 0)),
        ],
        out_shape=[
            jax.ShapeDtypeStruct((_N, 32), jnp.float32),
            jax.ShapeDtypeStruct((_N, 64), jnp.float32),
        ],
    )(acc2, haug2, asad2, b2.reshape(1, 32), fcW, fcb.reshape(1, 64))

    return (emb, out)


# speedup vs baseline: 1.1689x; 1.1689x over previous
import functools

import jax
import jax.numpy as jnp
from jax import lax
from jax.experimental import pallas as pl
from jax.experimental.pallas import tpu as pltpu
from jax.experimental.pallas import tpu_sc as plsc

_N = 10000
_E = 320000
_EPW = _E // 32
_K = 2
_BLK = _K * 128
_NFULL = _EPW // _BLK
_TAIL = _EPW - _NFULL * _BLK
_NACC = 10240
_BR = 2048
_GRID = 5


def _leaky(x):
    return jnp.where(x >= 0, x, 0.2 * x)


def _dense_in_body(x_ref, w_ref, asrc_ref, adst_ref, h_ref, asad_ref):
    h = jnp.dot(x_ref[...], w_ref[...], preferred_element_type=jnp.float32)
    h_ref[...] = h
    a_s = jnp.sum(h * asrc_ref[...], axis=1, keepdims=True)
    a_d = jnp.sum(h * adst_ref[...], axis=1, keepdims=True)
    asad_ref[...] = jnp.concatenate([a_s, a_d], axis=1)


def _combine_mid_body(acc_ref, den_ref, h_ref, asad_ref, b_ref, w_ref,
                      asrc_ref, adst_ref, h2_ref, asad2_ref):
    acc = acc_ref[0] + acc_ref[1]
    den = jnp.sum(den_ref[...], axis=0)
    asad = asad_ref[...]
    es = jnp.exp(_leaky(asad[:, 0] + asad[:, 1]))
    h1 = h_ref[...]
    num = acc + es[:, None] * h1
    den = den + es
    g = num / (den[:, None] + 1e-16) + b_ref[...]
    hl = jnp.maximum(g, 0.0)
    h2 = jnp.dot(hl, w_ref[...], preferred_element_type=jnp.float32)
    h2_ref[...] = h2
    a_s = jnp.sum(h2 * asrc_ref[...], axis=1, keepdims=True)
    a_d = jnp.sum(h2 * adst_ref[...], axis=1, keepdims=True)
    asad2_ref[...] = jnp.concatenate([a_s, a_d], axis=1)


def _combine_out_body(acc_ref, den_ref, h_ref, asad_ref, b_ref, w_ref,
                      fcb_ref, emb_ref, out_ref):
    acc = acc_ref[0] + acc_ref[1]
    den = jnp.sum(den_ref[...], axis=0)
    asad = asad_ref[...]
    es = jnp.exp(_leaky(asad[:, 0] + asad[:, 1]))
    h2 = h_ref[...]
    num = acc + es[:, None] * h2
    den = den + es
    g = num / (den[:, None] + 1e-16) + b_ref[...]
    emb = jnp.maximum(g, 0.0)
    emb_ref[...] = emb
    out_ref[...] = jnp.dot(emb, w_ref[...],
                           preferred_element_type=jnp.float32) + fcb_ref[...]


def _sc_edge_body(edge_hbm, asad_hbm, h_hbm, acc_hbm, den_hbm,
                  srcv, dstv, exv, rowsv, asadv, denv, acc_sh,
                  gsem, isem, ssem, *, dp):
    c = lax.axis_index("c")
    s = lax.axis_index("s")
    wid = s * 2 + c
    e0 = wid * _EPW
    nj = dp // 16

    pltpu.sync_copy(asad_hbm, asadv.at[pl.ds(0, 2 * _N)])

    @plsc.parallel_loop(0, _NACC // 16, unroll=8)
    def _zd(r):
        denv[pl.ds(r * 16, 16)] = jnp.zeros((16,), jnp.float32)

    @plsc.parallel_loop(0, 128, unroll=8)
    def _z(r):
        for j in range(nj):
            rowsv[0, r, pl.ds(j * 16, 16)] = jnp.zeros((16,), jnp.float32)
    for z in range(5):
        pltpu.sync_copy(rowsv.at[0, pl.ds(0, 128)],
                        acc_sh.at[pl.ds(s * 640 + z * 128, 128)])
    plsc.subcore_barrier()

    def _issue_idx(t, b):
        rb = e0 + t * _BLK
        for k in range(_K):
            pltpu.async_copy(edge_hbm.at[0, pl.ds(rb + k * 128, 128)],
                             srcv.at[b, k], isem)
            pltpu.async_copy(edge_hbm.at[1, pl.ds(rb + k * 128, 128)],
                             dstv.at[b, k], isem)

    def _wait_idx(b):
        for _ in range(2 * _K):
            pltpu.make_async_copy(edge_hbm.at[0, pl.ds(0, 128)],
                                  srcv.at[b, 0], isem).wait()

    def _wait_scatters(nk=_K):
        for k in range(nk):
            pltpu.make_async_copy(h_hbm.at[pl.ds(0, 128)],
                                  rowsv.at[0, pl.ds(0, 128)], ssem).wait()

    def _do_block(b, nk=_K):
        rows2 = rowsv.at[b]
        cps = [pltpu.async_copy(h_hbm.at[srcv.at[b, k]],
                                rows2.at[pl.ds(k * 128, 128)], gsem)
               for k in range(nk)]
        for k in range(nk):
            def _grp(i, _, k=k):
                sv = srcv[b, k, pl.ds(i * 16, 16)]
                dv = dstv[b, k, pl.ds(i * 16, 16)]
                a = plsc.load_gather(asadv, [sv * 2])
                ad = plsc.load_gather(asadv, [dv * 2 + 1])
                ex = jnp.exp(_leaky(a + ad))
                exv[pl.ds(k * 128 + i * 16, 16)] = ex
                plsc.addupdate_scatter(denv, [dv], ex)
                return 0
            lax.fori_loop(0, 8, _grp, 0)
        for cp in cps:
            cp.wait()

        @plsc.parallel_loop(0, nk * 128, unroll=8)
        def _scale(e):
            v = plsc.load_gather(exv, [jnp.full((16,), e, jnp.int32)])
            for j in range(nj):
                rows2[e, pl.ds(j * 16, 16)] = rows2[e, pl.ds(j * 16, 16)] * v

        for k in range(nk):
            pltpu.async_copy(rows2.at[pl.ds(k * 128, 128)],
                             acc_sh.at[dstv.at[b, k]], ssem, add=True)

    _issue_idx(0, 0)

    def _super(u, _):
        _wait_idx(0)

        @pl.when(u >= 1)
        def _():
            _wait_scatters()
        _issue_idx(2 * u + 1, 1)
        _do_block(0)

        _wait_idx(1)
        _wait_scatters()
        _issue_idx(2 * u + 2, 0)
        _do_block(1)
        return 0
    lax.fori_loop(0, (_NFULL - 1) // 2, _super, 0)

    _wait_idx(0)
    _wait_scatters()
    _do_block(0)

    pltpu.sync_copy(edge_hbm.at[0, pl.ds(e0 + _NFULL * _BLK, _TAIL)],
                    srcv.at[1, 0, pl.ds(0, _TAIL)])
    pltpu.sync_copy(edge_hbm.at[1, pl.ds(e0 + _NFULL * _BLK, _TAIL)],
                    dstv.at[1, 0, pl.ds(0, _TAIL)])
    iota16 = lax.iota(jnp.int32, 16)
    for g in range(_TAIL // 16, 8):
        srcv[1, 0, pl.ds(g * 16, 16)] = ((iota16 + g * 16) * 613) % _N
        dstv[1, 0, pl.ds(g * 16, 16)] = _N + ((iota16 + g * 16 + wid * 8)
                                              % (_NACC - _N))
    _wait_scatters()
    _do_block(1, nk=1)
    _wait_scatters(nk=1)
    plsc.subcore_barrier()

    pltpu.sync_copy(acc_sh.at[pl.ds(s * 640, 640)],
                    acc_hbm.at[c, pl.ds(s * 640, 640)])
    pltpu.sync_copy(denv, den_hbm.at[wid])


def _sc_edge(edge_index, asad, h, dp):
    mesh = plsc.VectorSubcoreMesh(core_axis_name="c", subcore_axis_name="s",
                                  num_cores=2, num_subcores=16)
    f = pl.kernel(
        functools.partial(_sc_edge_body, dp=dp),
        out_type=[
            jax.ShapeDtypeStruct((2, _NACC, dp), jnp.float32),
            jax.ShapeDtypeStruct((32, _NACC), jnp.float32),
        ],
        mesh=mesh,
        compiler_params=pltpu.CompilerParams(needs_layout_passes=False,
                                             use_tc_tiling_on_sc=False),
        scratch_types=[
            pltpu.VMEM((2, _K, 128), jnp.int32),
            pltpu.VMEM((2, _K, 128), jnp.int32),
            pltpu.VMEM((_BLK,), jnp.float32),
            pltpu.VMEM((2, _BLK, dp), jnp.float32),
            pltpu.VMEM((2 * _NACC,), jnp.float32),
            pltpu.VMEM((_NACC,), jnp.float32),
            pltpu.VMEM_SHARED((_NACC, dp), jnp.float32),
            pltpu.SemaphoreType.DMA,
            pltpu.SemaphoreType.DMA,
            pltpu.SemaphoreType.DMA,
        ],
    )
    return f(edge_index, asad.reshape(-1), h)


def _full_spec(shape):
    return pl.BlockSpec(shape, lambda i, _z=len(shape): (0,) * _z)


def kernel(x, edge_index, W1, a_src1, a_dst1, b1, W2, a_src2, a_dst2, b2,
           fcW, fcb):
    h1, asad1 = pl.pallas_call(
        _dense_in_body,
        grid=(_GRID,),
        in_specs=[
            pl.BlockSpec((_BR, 128), lambda i: (i, 0)),
            _full_spec((128, 64)),
            _full_spec((1, 64)),
            _full_spec((1, 64)),
        ],
        out_specs=[
            pl.BlockSpec((_BR, 64), lambda i: (i, 0)),
            pl.BlockSpec((_BR, 2), lambda i: (i, 0)),
        ],
        out_shape=[
            jax.ShapeDtypeStruct((_N, 64), jnp.float32),
            jax.ShapeDtypeStruct((_N, 2), jnp.float32),
        ],
    )(x, W1, a_src1.reshape(1, 64), a_dst1.reshape(1, 64))

    acc1, den1 = _sc_edge(edge_index, asad1, h1, 64)

    h2, asad2 = pl.pallas_call(
        _combine_mid_body,
        grid=(_GRID,),
        in_specs=[
            pl.BlockSpec((2, _BR, 64), lambda i: (0, i, 0)),
            pl.BlockSpec((32, _BR), lambda i: (0, i)),
            pl.BlockSpec((_BR, 64), lambda i: (i, 0)),
            pl.BlockSpec((_BR, 2), lambda i: (i, 0)),
            _full_spec((1, 64)),
            _full_spec((64, 32)),
            _full_spec((1, 32)),
            _full_spec((1, 32)),
        ],
        out_specs=[
            pl.BlockSpec((_BR, 32), lambda i: (i, 0)),
            pl.BlockSpec((_BR, 2), lambda i: (i, 0)),
        ],
        out_shape=[
            jax.ShapeDtypeStruct((_N, 32), jnp.float32),
            jax.ShapeDtypeStruct((_N, 2), jnp.float32),
        ],
    )(acc1, den1, h1, asad1, b1.reshape(1, 64), W2,
      a_src2.reshape(1, 32), a_dst2.reshape(1, 32))

    acc2, den2 = _sc_edge(edge_index, asad2, h2, 32)

    emb, out = pl.pallas_call(
        _combine_out_body,
        grid=(_GRID,),
        in_specs=[
            pl.BlockSpec((2, _BR, 32), lambda i: (0, i, 0)),
            pl.BlockSpec((32, _BR), lambda i: (0, i)),
            pl.BlockSpec((_BR, 32), lambda i: (i, 0)),
            pl.BlockSpec((_BR, 2), lambda i: (i, 0)),
            _full_spec((1, 32)),
            _full_spec((32, 64)),
            _full_spec((1, 64)),
        ],
        out_specs=[
            pl.BlockSpec((_BR, 32), lambda i: (i, 0)),
            pl.BlockSpec((_BR, 64), lambda i: (i, 0)),
        ],
        out_shape=[
            jax.ShapeDtypeStruct((_N, 32), jnp.float32),
            jax.ShapeDtypeStruct((_N, 64), jnp.float32),
        ],
    )(acc2, den2, h2, asad2, b2.reshape(1, 32), fcW, fcb.reshape(1, 64))

    return (emb, out)


# speedup vs baseline: 1.2306x; 1.0528x over previous
import functools

import jax
import jax.numpy as jnp
from jax import lax
from jax.experimental import pallas as pl
from jax.experimental.pallas import tpu as pltpu
from jax.experimental.pallas import tpu_sc as plsc

_N = 10000
_E = 320000
_EPW = _E // 32
_K = 3
_BLK = _K * 128
_NFULL = _EPW // _BLK
_TAIL = _EPW - _NFULL * _BLK
_NACC = 10240
_BR = 2048
_GRID = 5


def _leaky(x):
    return jnp.where(x >= 0, x, 0.2 * x)


def _dense_in_body(x_ref, w_ref, asrc_ref, adst_ref, h_ref, asad_ref):
    h = jnp.dot(x_ref[...], w_ref[...], preferred_element_type=jnp.float32)
    h_ref[...] = h
    a_s = jnp.sum(h * asrc_ref[...], axis=1, keepdims=True)
    a_d = jnp.sum(h * adst_ref[...], axis=1, keepdims=True)
    asad_ref[...] = jnp.concatenate([a_s, a_d], axis=1)


def _combine_mid_body(acc_ref, den_ref, h_ref, asad_ref, b_ref, w_ref,
                      asrc_ref, adst_ref, h2_ref, asad2_ref):
    acc = acc_ref[0] + acc_ref[1]
    den = jnp.sum(den_ref[...], axis=0)
    asad = asad_ref[...]
    es = jnp.exp(_leaky(asad[:, 0] + asad[:, 1]))
    h1 = h_ref[...]
    num = acc + es[:, None] * h1
    den = den + es
    g = num / (den[:, None] + 1e-16) + b_ref[...]
    hl = jnp.maximum(g, 0.0)
    h2 = jnp.dot(hl, w_ref[...], preferred_element_type=jnp.float32)
    h2_ref[...] = h2
    a_s = jnp.sum(h2 * asrc_ref[...], axis=1, keepdims=True)
    a_d = jnp.sum(h2 * adst_ref[...], axis=1, keepdims=True)
    asad2_ref[...] = jnp.concatenate([a_s, a_d], axis=1)


def _combine_out_body(acc_ref, den_ref, h_ref, asad_ref, b_ref, w_ref,
                      fcb_ref, emb_ref, out_ref):
    acc = acc_ref[0] + acc_ref[1]
    den = jnp.sum(den_ref[...], axis=0)
    asad = asad_ref[...]
    es = jnp.exp(_leaky(asad[:, 0] + asad[:, 1]))
    h2 = h_ref[...]
    num = acc + es[:, None] * h2
    den = den + es
    g = num / (den[:, None] + 1e-16) + b_ref[...]
    emb = jnp.maximum(g, 0.0)
    emb_ref[...] = emb
    out_ref[...] = jnp.dot(emb, w_ref[...],
                           preferred_element_type=jnp.float32) + fcb_ref[...]


def _sc_edge_body(edge_hbm, asad_hbm, h_hbm, acc_hbm, den_hbm,
                  srcv, dstv, exv, rowsv, asadv, denv, acc_sh,
                  gsem, isem, ssem, *, dp):
    c = lax.axis_index("c")
    s = lax.axis_index("s")
    wid = s * 2 + c
    e0 = wid * _EPW
    nj = dp // 16

    pltpu.sync_copy(asad_hbm, asadv.at[pl.ds(0, 2 * _N)])

    @plsc.parallel_loop(0, _NACC // 16, unroll=8)
    def _zd(r):
        denv[pl.ds(r * 16, 16)] = jnp.zeros((16,), jnp.float32)

    @plsc.parallel_loop(0, 128, unroll=8)
    def _z(r):
        for j in range(nj):
            rowsv[0, r, pl.ds(j * 16, 16)] = jnp.zeros((16,), jnp.float32)
    for z in range(5):
        pltpu.sync_copy(rowsv.at[0, pl.ds(0, 128)],
                        acc_sh.at[pl.ds(s * 640 + z * 128, 128)])
    plsc.subcore_barrier()

    def _issue_idx(t, b):
        rb = e0 + t * _BLK
        for k in range(_K):
            pltpu.async_copy(edge_hbm.at[0, pl.ds(rb + k * 128, 128)],
                             srcv.at[b, k], isem)
            pltpu.async_copy(edge_hbm.at[1, pl.ds(rb + k * 128, 128)],
                             dstv.at[b, k], isem)

    def _wait_idx(b):
        for _ in range(2 * _K):
            pltpu.make_async_copy(edge_hbm.at[0, pl.ds(0, 128)],
                                  srcv.at[b, 0], isem).wait()

    def _wait_scatters(nk=_K):
        for k in range(nk):
            pltpu.make_async_copy(h_hbm.at[pl.ds(0, 128)],
                                  rowsv.at[0, pl.ds(0, 128)], ssem).wait()

    def _do_block(b, nk=_K):
        rows2 = rowsv.at[b]
        cps = [pltpu.async_copy(h_hbm.at[srcv.at[b, k]],
                                rows2.at[pl.ds(k * 128, 128)], gsem)
               for k in range(nk)]
        for k in range(nk):
            def _grp(i, _, k=k):
                sv = srcv[b, k, pl.ds(i * 16, 16)]
                dv = dstv[b, k, pl.ds(i * 16, 16)]
                a = plsc.load_gather(asadv, [sv * 2])
                ad = plsc.load_gather(asadv, [dv * 2 + 1])
                ex = jnp.exp(_leaky(a + ad))
                exv[pl.ds(k * 128 + i * 16, 16)] = ex
                plsc.addupdate_scatter(denv, [dv], ex)
                return 0
            lax.fori_loop(0, 8, _grp, 0)
        for cp in cps:
            cp.wait()

        @plsc.parallel_loop(0, nk * 128, unroll=8)
        def _scale(e):
            v = plsc.load_gather(exv, [jnp.full((16,), e, jnp.int32)])
            for j in range(nj):
                rows2[e, pl.ds(j * 16, 16)] = rows2[e, pl.ds(j * 16, 16)] * v

        for k in range(nk):
            pltpu.async_copy(rows2.at[pl.ds(k * 128, 128)],
                             acc_sh.at[dstv.at[b, k]], ssem, add=True)

    _issue_idx(0, 0)

    def _super(u, _):
        _wait_idx(0)

        @pl.when(u >= 1)
        def _():
            _wait_scatters()
        _issue_idx(2 * u + 1, 1)
        _do_block(0)

        _wait_idx(1)
        _wait_scatters()
        _issue_idx(2 * u + 2, 0)
        _do_block(1)
        return 0
    nsup = (_NFULL - 1) // 2
    lax.fori_loop(0, nsup, _super, 0)

    for tb in range(2 * nsup, _NFULL):
        bb = tb % 2
        _wait_idx(bb)
        _wait_scatters()
        if tb + 1 < _NFULL:
            _issue_idx(tb + 1, bb ^ 1)
        _do_block(bb)

    tbuf = _NFULL % 2
    pltpu.sync_copy(edge_hbm.at[0, pl.ds(e0 + _NFULL * _BLK, _TAIL)],
                    srcv.at[tbuf, 0, pl.ds(0, _TAIL)])
    pltpu.sync_copy(edge_hbm.at[1, pl.ds(e0 + _NFULL * _BLK, _TAIL)],
                    dstv.at[tbuf, 0, pl.ds(0, _TAIL)])
    iota16 = lax.iota(jnp.int32, 16)
    for g in range(_TAIL // 16, 8):
        srcv[tbuf, 0, pl.ds(g * 16, 16)] = ((iota16 + g * 16) * 613) % _N
        dstv[tbuf, 0, pl.ds(g * 16, 16)] = _N + ((iota16 + g * 16 + wid * 8)
                                                 % (_NACC - _N))
    _wait_scatters()
    _do_block(tbuf, nk=1)
    _wait_scatters(nk=1)
    plsc.subcore_barrier()

    pltpu.sync_copy(acc_sh.at[pl.ds(s * 640, 640)],
                    acc_hbm.at[c, pl.ds(s * 640, 640)])
    pltpu.sync_copy(denv, den_hbm.at[wid])


def _sc_edge(edge_index, asad, h, dp):
    mesh = plsc.VectorSubcoreMesh(core_axis_name="c", subcore_axis_name="s",
                                  num_cores=2, num_subcores=16)
    f = pl.kernel(
        functools.partial(_sc_edge_body, dp=dp),
        out_type=[
            jax.ShapeDtypeStruct((2, _NACC, dp), jnp.float32),
            jax.ShapeDtypeStruct((32, _NACC), jnp.float32),
        ],
        mesh=mesh,
        compiler_params=pltpu.CompilerParams(needs_layout_passes=False,
                                             use_tc_tiling_on_sc=False),
        scratch_types=[
            pltpu.VMEM((2, _K, 128), jnp.int32),
            pltpu.VMEM((2, _K, 128), jnp.int32),
            pltpu.VMEM((_BLK,), jnp.float32),
            pltpu.VMEM((2, _BLK, dp), jnp.float32),
            pltpu.VMEM((2 * _NACC,), jnp.float32),
            pltpu.VMEM((_NACC,), jnp.float32),
            pltpu.VMEM_SHARED((_NACC, dp), jnp.float32),
            pltpu.SemaphoreType.DMA,
            pltpu.SemaphoreType.DMA,
            pltpu.SemaphoreType.DMA,
        ],
    )
    return f(edge_index, asad.reshape(-1), h)


def _full_spec(shape):
    return pl.BlockSpec(shape, lambda i, _z=len(shape): (0,) * _z)


def kernel(x, edge_index, W1, a_src1, a_dst1, b1, W2, a_src2, a_dst2, b2,
           fcW, fcb):
    h1, asad1 = pl.pallas_call(
        _dense_in_body,
        grid=(_GRID,),
        in_specs=[
            pl.BlockSpec((_BR, 128), lambda i: (i, 0)),
            _full_spec((128, 64)),
            _full_spec((1, 64)),
            _full_spec((1, 64)),
        ],
        out_specs=[
            pl.BlockSpec((_BR, 64), lambda i: (i, 0)),
            pl.BlockSpec((_BR, 2), lambda i: (i, 0)),
        ],
        out_shape=[
            jax.ShapeDtypeStruct((_N, 64), jnp.float32),
            jax.ShapeDtypeStruct((_N, 2), jnp.float32),
        ],
    )(x, W1, a_src1.reshape(1, 64), a_dst1.reshape(1, 64))

    acc1, den1 = _sc_edge(edge_index, asad1, h1, 64)

    h2, asad2 = pl.pallas_call(
        _combine_mid_body,
        grid=(_GRID,),
        in_specs=[
            pl.BlockSpec((2, _BR, 64), lambda i: (0, i, 0)),
            pl.BlockSpec((32, _BR), lambda i: (0, i)),
            pl.BlockSpec((_BR, 64), lambda i: (i, 0)),
            pl.BlockSpec((_BR, 2), lambda i: (i, 0)),
            _full_spec((1, 64)),
            _full_spec((64, 32)),
            _full_spec((1, 32)),
            _full_spec((1, 32)),
        ],
        out_specs=[
            pl.BlockSpec((_BR, 32), lambda i: (i, 0)),
            pl.BlockSpec((_BR, 2), lambda i: (i, 0)),
        ],
        out_shape=[
            jax.ShapeDtypeStruct((_N, 32), jnp.float32),
            jax.ShapeDtypeStruct((_N, 2), jnp.float32),
        ],
    )(acc1, den1, h1, asad1, b1.reshape(1, 64), W2,
      a_src2.reshape(1, 32), a_dst2.reshape(1, 32))

    acc2, den2 = _sc_edge(edge_index, asad2, h2, 32)

    emb, out = pl.pallas_call(
        _combine_out_body,
        grid=(_GRID,),
        in_specs=[
            pl.BlockSpec((2, _BR, 32), lambda i: (0, i, 0)),
            pl.BlockSpec((32, _BR), lambda i: (0, i)),
            pl.BlockSpec((_BR, 32), lambda i: (i, 0)),
            pl.BlockSpec((_BR, 2), lambda i: (i, 0)),
            _full_spec((1, 32)),
            _full_spec((32, 64)),
            _full_spec((1, 64)),
        ],
        out_specs=[
            pl.BlockSpec((_BR, 32), lambda i: (i, 0)),
            pl.BlockSpec((_BR, 64), lambda i: (i, 0)),
        ],
        out_shape=[
            jax.ShapeDtypeStruct((_N, 32), jnp.float32),
            jax.ShapeDtypeStruct((_N, 64), jnp.float32),
        ],
    )(acc2, den2, h2, asad2, b2.reshape(1, 32), fcW, fcb.reshape(1, 64))

    return (emb, out)
